# baseline (speedup 1.0000x reference)
import jax
import jax.numpy as jnp
from jax import lax
from jax.experimental import pallas as pl
from jax.experimental.pallas import tpu as pltpu
from jax.experimental.pallas import tpu_sc as plsc
import functools

NUM_FRAMES = 1024
NUM_TYPES = 4
NUM_ACTIONS = 128
TOTAL_CLS = 256
FB = 64


def _onehot_body(a_ref, o_ref):
    a = a_ref[...]
    toff = lax.broadcasted_iota(jnp.int32, (FB, NUM_TYPES, NUM_ACTIONS), 1)
    a2 = (a + (toff << 6)).astype(jnp.float32)
    t_io = lax.broadcasted_iota(jnp.int32, (NUM_TYPES, TOTAL_CLS), 0)
    c_io = lax.broadcasted_iota(jnp.int32, (NUM_TYPES, TOTAL_CLS), 1)
    p = (t_io == (c_io >> 6)).astype(jnp.float32)
    k = lax.dot_general(
        a2, p, (((1,), (0,)), ((), ())), preferred_element_type=jnp.float32
    )
    col = lax.broadcasted_iota(
        jnp.int32, (FB, NUM_ACTIONS, TOTAL_CLS), 2
    ).astype(jnp.float32)
    o_ref[...] = (k == col).astype(jnp.float32)


def _kernel_tc(actions):
    grid = (NUM_FRAMES // FB,)
    return pl.pallas_call(
        _onehot_body,
        grid=grid,
        in_specs=[
            pl.BlockSpec((FB, NUM_TYPES, NUM_ACTIONS), lambda i: (i, 0, 0))
        ],
        out_specs=pl.BlockSpec(
            (FB, NUM_ACTIONS, TOTAL_CLS), lambda i: (i, 0, 0)
        ),
        out_shape=jax.ShapeDtypeStruct(
            (NUM_FRAMES, NUM_ACTIONS, TOTAL_CLS), jnp.float32
        ),
    )(actions)


ROW = NUM_ACTIONS * TOTAL_CLS
NW = 32
FPW = NUM_FRAMES // NW
AW = NUM_TYPES * NUM_ACTIONS


NBUF = 3


def _sc_body(a_hbm, z_hbm, out_hbm, shared, s0, s1, s2):
    wid = lax.axis_index("s") * 2 + lax.axis_index("c")
    fbase = wid * FPW
    sems = (s0, s1)
    for b in range(2):
        pltpu.async_copy(
            shared, out_hbm.at[pl.ds(fbase + 16 * b, 16)], sems[b]
        )
    for b in range(2):
        pltpu.make_async_copy(
            shared, out_hbm.at[pl.ds(fbase + 16 * b, 16)], sems[b]
        ).wait()


def _kernel_sc(actions):
    mesh = plsc.VectorSubcoreMesh(core_axis_name="c", subcore_axis_name="s")
    sck = functools.partial(
        pl.kernel,
        out_type=jax.ShapeDtypeStruct((NUM_FRAMES, ROW), jnp.float32),
        mesh=mesh,
        scratch_types=[
            pltpu.VMEM_SHARED((16, ROW), jnp.float32),
            pltpu.SemaphoreType.DMA,
            pltpu.SemaphoreType.DMA,
            pltpu.SemaphoreType.DMA,
        ],
        compiler_params=pltpu.CompilerParams(needs_layout_passes=False),
    )(_sc_body)
    af = actions.reshape(NUM_FRAMES * AW)
    zrow = jnp.zeros((ROW,), jnp.float32)
    out = sck(af, zrow)
    return out.reshape(NUM_FRAMES, NUM_ACTIONS, TOTAL_CLS)


kernel = _kernel_sc


# speedup vs baseline: 1.0419x; 1.0419x over previous
import functools

import jax
import jax.numpy as jnp
from jax import lax
from jax.experimental import pallas as pl
from jax.experimental.pallas import tpu as pltpu
from jax.experimental.pallas import tpu_sc as plsc

NUM_FRAMES = 1024
NUM_TYPES = 4
NUM_ACTIONS = 128
TOTAL_CLS = 256
ROW = NUM_ACTIONS * TOTAL_CLS
NW = 32
FPW = NUM_FRAMES // NW
AW = NUM_TYPES * NUM_ACTIONS
NBUF = 2


def _sc_body(a_hbm, z_hbm, out_hbm, a_v, b0, b1, s0, s1):
    wid = lax.axis_index("s") * 2 + lax.axis_index("c")
    fbase = wid * FPW
    pltpu.sync_copy(a_hbm.at[pl.ds(fbase * AW, FPW * AW)], a_v)
    bufs = (b0, b1)
    sems = (s0, s1)
    for b in range(NBUF):
        pltpu.sync_copy(z_hbm, bufs[b])

    siota = lax.iota(jnp.int32, 16) * TOTAL_CLS
    ones = jnp.ones((16,), jnp.float32)
    zeros = jnp.zeros((16,), jnp.float32)

    def put(buf, i, val):
        for t in range(NUM_TYPES):
            for ch in range(NUM_ACTIONS // 16):
                av = a_v[pl.ds(i * AW + t * NUM_ACTIONS + ch * 16, 16)]
                idx = av + (siota + (ch * 16 * TOTAL_CLS + t * 64))
                plsc.store_scatter(buf, [idx], val)

    def advance(b, i):
        pltpu.make_async_copy(
            bufs[b], out_hbm.at[fbase + i - NBUF], sems[b]
        ).wait()
        put(bufs[b], i - NBUF, zeros)
        put(bufs[b], i, ones)
        pltpu.async_copy(bufs[b], out_hbm.at[fbase + i], sems[b])

    for b in range(NBUF):
        put(bufs[b], b, ones)
        pltpu.async_copy(bufs[b], out_hbm.at[fbase + b], sems[b])

    def step(k, _):
        g = NBUF * k
        for b in range(NBUF):
            advance(b, g + b)
        return _

    nfull = (FPW - NBUF) // NBUF
    lax.fori_loop(1, 1 + nfull, step, 0)

    done = NBUF + nfull * NBUF
    for i in range(done, FPW):
        advance(i % NBUF, i)
    for i in range(FPW - NBUF, FPW):
        pltpu.make_async_copy(
            bufs[i % NBUF], out_hbm.at[fbase + i], sems[i % NBUF]
        ).wait()


def kernel(actions):
    mesh = plsc.VectorSubcoreMesh(core_axis_name="c", subcore_axis_name="s")
    sck = functools.partial(
        pl.kernel,
        out_type=jax.ShapeDtypeStruct((NUM_FRAMES, ROW), jnp.float32),
        mesh=mesh,
        scratch_types=[
            pltpu.VMEM((FPW * AW,), jnp.int32),
            pltpu.VMEM((ROW,), jnp.float32),
            pltpu.VMEM((ROW,), jnp.float32),
            pltpu.SemaphoreType.DMA,
            pltpu.SemaphoreType.DMA,
        ],
        compiler_params=pltpu.CompilerParams(needs_layout_passes=False),
    )(_sc_body)
    af = actions.reshape(NUM_FRAMES * AW)
    zrow = jnp.zeros((ROW,), jnp.float32)
    out = sck(af, zrow)
    return out.reshape(NUM_FRAMES, NUM_ACTIONS, TOTAL_CLS)
